# baseline (speedup 1.0000x reference)
import functools

import jax
import jax.numpy as jnp
from jax import lax
from jax.experimental import pallas as pl
from jax.experimental.pallas import tpu as pltpu
from jax.experimental.pallas import tpu_sc as plsc


def _make_selector(B, S, D):
    R = S // 2
    G = B * R
    NW = 32
    rows_per_w = G // NW

    C = 8
    NBUF = 15
    n_chunks = rows_per_w // C

    mesh = plsc.VectorSubcoreMesh(core_axis_name="c", subcore_axis_name="s")

    @functools.partial(
        pl.kernel,
        mesh=mesh,
        out_type=jax.ShapeDtypeStruct((G, D), jnp.float32),
        scratch_types=(
            [pltpu.VMEM((rows_per_w,), jnp.int32)]
            + [pltpu.VMEM((C, D), jnp.float32) for _ in range(NBUF)]
            + [pltpu.SemaphoreType.DMA for _ in range(2 * NBUF)]
        ),
    )
    def run(x_hbm, out_hbm, idxv, *scratch):
        bufs = scratch[:NBUF]
        lsems = scratch[NBUF:2 * NBUF]
        ssems = scratch[2 * NBUF:]
        wid = lax.axis_index("s") * 2 + lax.axis_index("c")
        base = wid * rows_per_w

        iota2 = lax.iota(jnp.int32, 16) * 2
        first = 2 * base + 1
        for k in range(rows_per_w // 16):
            idxv[pl.ds(k * 16, 16)] = iota2 + (first + 32 * k)

        def load(g):
            b = g % NBUF
            return pltpu.async_copy(
                x_hbm.at[idxv.at[pl.ds(g * C, C)]], bufs[b], lsems[b])

        def store(g):
            b = g % NBUF
            return pltpu.async_copy(
                bufs[b], out_hbm.at[pl.ds(base + g * C, C), :], ssems[b])

        ld = {0: load(0)}
        st = {}
        for g in range(n_chunks):
            nxt = g + 1
            if nxt < n_chunks:
                if nxt >= NBUF:
                    st[nxt - NBUF].wait()
                ld[nxt] = load(nxt)
            ld[g].wait()
            st[g] = store(g)
        for g in range(max(0, n_chunks - NBUF), n_chunks):
            st[g].wait()

    return run


def kernel(inputs):
    B, S, D = inputs.shape
    x2 = inputs.reshape(B * S, D)
    out = _make_selector(B, S, D)(x2)
    return out.reshape(B, S // 2, D)


# speedup vs baseline: 1.1037x; 1.1037x over previous
import functools

import jax
import jax.numpy as jnp
from jax import lax
from jax.experimental import pallas as pl
from jax.experimental.pallas import tpu as pltpu
from jax.experimental.pallas import tpu_sc as plsc


def _make_selector(B, S, D):
    R = S // 2
    G = B * R
    NW = 32
    rows_per_w = G // NW

    C = 16
    NBUF = 7
    n_chunks = rows_per_w // C

    mesh = plsc.VectorSubcoreMesh(core_axis_name="c", subcore_axis_name="s")

    @functools.partial(
        pl.kernel,
        mesh=mesh,
        out_type=jax.ShapeDtypeStruct((G, D), jnp.float32),
        scratch_types=(
            [pltpu.VMEM((rows_per_w,), jnp.int32)]
            + [pltpu.VMEM((C, D), jnp.float32) for _ in range(NBUF)]
            + [pltpu.SemaphoreType.DMA for _ in range(2 * NBUF)]
        ),
    )
    def run(x_hbm, out_hbm, idxv, *scratch):
        bufs = scratch[:NBUF]
        lsems = scratch[NBUF:2 * NBUF]
        ssems = scratch[2 * NBUF:]
        wid = lax.axis_index("s") * 2 + lax.axis_index("c")
        base = wid * rows_per_w

        iota2 = lax.iota(jnp.int32, 16) * 2
        first = 2 * base + 1
        for k in range(rows_per_w // 16):
            idxv[pl.ds(k * 16, 16)] = iota2 + (first + 32 * k)

        def load(g):
            b = g % NBUF
            return pltpu.async_copy(
                x_hbm.at[idxv.at[pl.ds(g * C, C)]], bufs[b], lsems[b])

        def store(g):
            b = g % NBUF
            return pltpu.async_copy(
                bufs[b], out_hbm.at[pl.ds(base + g * C, C), :], ssems[b])

        LA = 3
        ld = {}
        st = {}

        def issue_load(j):
            if j >= NBUF:
                st[j - NBUF].wait()
            ld[j] = load(j)

        for j in range(min(LA, n_chunks)):
            issue_load(j)
        for g in range(n_chunks):
            if g + LA < n_chunks:
                issue_load(g + LA)
            ld[g].wait()
            st[g] = store(g)
        for g in range(max(0, n_chunks - NBUF), n_chunks):
            st[g].wait()

    return run


def kernel(inputs):
    B, S, D = inputs.shape
    x2 = inputs.reshape(B * S, D)
    out = _make_selector(B, S, D)(x2)
    return out.reshape(B, S // 2, D)


# speedup vs baseline: 1.1053x; 1.0015x over previous
import functools

import jax
import jax.numpy as jnp
from jax import lax
from jax.experimental import pallas as pl
from jax.experimental.pallas import tpu as pltpu
from jax.experimental.pallas import tpu_sc as plsc


def _make_selector(B, S, D):
    R = S // 2
    G = B * R
    NW = 32
    rows_per_w = G // NW

    C = 16
    NBUF = 7
    n_chunks = rows_per_w // C

    mesh = plsc.VectorSubcoreMesh(core_axis_name="c", subcore_axis_name="s")

    @functools.partial(
        pl.kernel,
        mesh=mesh,
        out_type=jax.ShapeDtypeStruct((G, D), jnp.float32),
        scratch_types=(
            [pltpu.VMEM((rows_per_w,), jnp.int32)]
            + [pltpu.VMEM((C, D), jnp.float32) for _ in range(NBUF)]
            + [pltpu.SemaphoreType.DMA for _ in range(2 * NBUF)]
        ),
    )
    def run(x_hbm, out_hbm, idxv, *scratch):
        bufs = scratch[:NBUF]
        lsems = scratch[NBUF:2 * NBUF]
        ssems = scratch[2 * NBUF:]
        wid = lax.axis_index("s") * 2 + lax.axis_index("c")
        base = wid * rows_per_w

        iota2 = lax.iota(jnp.int32, 16) * 2
        first = 2 * base + 1
        for k in range(rows_per_w // 16):
            idxv[pl.ds(k * 16, 16)] = iota2 + (first + 32 * k)

        def load(g):
            b = g % NBUF
            return pltpu.async_copy(
                x_hbm.at[idxv.at[pl.ds(g * C, C)]], bufs[b], lsems[b])

        def store(g):
            b = g % NBUF
            return pltpu.async_copy(
                bufs[b], out_hbm.at[pl.ds(base + g * C, C), :], ssems[b])

        LA = 5
        ld = {}
        st = {}

        def issue_load(j):
            if j >= NBUF:
                st[j - NBUF].wait()
            ld[j] = load(j)

        for j in range(min(LA, n_chunks)):
            issue_load(j)
        for g in range(n_chunks):
            if g + LA < n_chunks:
                issue_load(g + LA)
            ld[g].wait()
            st[g] = store(g)
        for g in range(max(0, n_chunks - NBUF), n_chunks):
            st[g].wait()

    return run


def kernel(inputs):
    B, S, D = inputs.shape
    x2 = inputs.reshape(B * S, D)
    out = _make_selector(B, S, D)(x2)
    return out.reshape(B, S // 2, D)
